# baseline (speedup 1.0000x reference)
import numpy as np

import jax
import jax.numpy as jnp
from jax.experimental import pallas as pl
from jax.experimental.pallas import tpu as pltpu

_N_TERMS = 64
_N_OUT = 4
_MASKS = [np.ones(_N_TERMS, dtype=bool) for _ in range(_N_OUT)]

_C = 16128
_K = 8
_A = 4


def _chunks(n):
    out, off = [], 0
    n_main = (n // _C) * _C
    while off < n_main:
        out.append((off, _C))
        off += _C
    rem = ((n - n_main) // 128) * 128
    if rem:
        out.append((n_main, rem))
    return out


def _make_copy_kernel(n):
    chunks = _chunks(n)
    nc = len(chunks)

    def body(x_ref, o_ref, buf, in_sems, out_sems):
        def in_copy(i):
            off, sz = chunks[i]
            s = i % _K
            return pltpu.make_async_copy(
                x_ref.at[:, pl.ds(off, sz)],
                buf.at[s, :, pl.ds(0, sz)],
                in_sems.at[s],
            )

        def out_copy(i):
            off, sz = chunks[i]
            s = i % _K
            return pltpu.make_async_copy(
                buf.at[s, :, pl.ds(0, sz)],
                o_ref.at[:, pl.ds(off, sz)],
                out_sems.at[s],
            )

        for i in range(min(_A, nc)):
            in_copy(i).start()
        for i in range(nc):
            j = i + _A
            if j < nc:
                if j >= _K:
                    out_copy(j - _K).wait()
                in_copy(j).start()
            in_copy(i).wait()
            out_copy(i).start()
        for i in range(max(0, nc - _K), nc):
            out_copy(i).wait()

    return body


def _tail_kernel(prev_ref, x_ref, o_ref):
    del prev_ref
    o_ref[...] = x_ref[...]


def _masked_gather_t(thetas_t, rows):
    w, n = thetas_t.shape
    main = pl.pallas_call(
        _make_copy_kernel(n),
        in_specs=[pl.BlockSpec(memory_space=pl.ANY)],
        out_specs=pl.BlockSpec(memory_space=pl.ANY),
        out_shape=jax.ShapeDtypeStruct((w, n), thetas_t.dtype),
        scratch_shapes=[
            pltpu.VMEM((_K, w, _C), thetas_t.dtype),
            pltpu.SemaphoreType.DMA((_K,)),
            pltpu.SemaphoreType.DMA((_K,)),
        ],
    )(thetas_t)
    covered = sum(sz for _, sz in _chunks(n))
    if covered == n:
        return main
    tb = covered // 128
    return pl.pallas_call(
        _tail_kernel,
        grid=(1,),
        in_specs=[
            pl.BlockSpec(memory_space=pl.ANY),
            pl.BlockSpec((w, 128), lambda i: (0, tb)),
        ],
        out_specs=pl.BlockSpec((w, 128), lambda i: (0, tb)),
        out_shape=jax.ShapeDtypeStruct((w, n), thetas_t.dtype),
        input_output_aliases={0: 0},
    )(main, thetas_t)


def kernel(thetas, time_derivs, coeff_0, coeff_1, coeff_2, coeff_3):
    rows = np.nonzero(_MASKS[0])[0].astype(np.int32)
    gathered = _masked_gather_t(thetas.T, rows).T
    sparse_thetas = (gathered,) * _N_OUT
    return sparse_thetas + (coeff_0, coeff_1, coeff_2, coeff_3)
